# baseline (speedup 1.0000x reference)
import functools

import jax
import jax.numpy as jnp
from jax import lax
from jax.experimental import pallas as pl
from jax.experimental.pallas import tpu as pltpu
from jax.experimental.pallas import tpu_sc as plsc

B = 16384
D = 64
NC = 2
NS = 16
NW = NC * NS
BW = B // NW
L = 16
CH = 128
NCH = BW // CH


def _sc_gather_partials(uidx, fidx, uemb2, user_bias, femb2, food_bias):
    mesh = plsc.VectorSubcoreMesh(core_axis_name="c", subcore_axis_name="s")

    @functools.partial(
        pl.kernel,
        mesh=mesh,
        out_type=(
            jax.ShapeDtypeStruct((NW * L,), jnp.float32),
            jax.ShapeDtypeStruct((B,), jnp.float32),
            jax.ShapeDtypeStruct((B,), jnp.float32),
        ),
        scratch_types=[
            pltpu.VMEM((BW,), jnp.int32),
            pltpu.VMEM((BW,), jnp.int32),
            pltpu.VMEM((BW,), jnp.int32),
            pltpu.VMEM((BW,), jnp.int32),
            pltpu.VMEM((BW,), jnp.int32),
            pltpu.VMEM((BW,), jnp.int32),
            pltpu.VMEM((2, CH, 2 * D), jnp.float32),
            pltpu.VMEM((2, CH, 2 * D), jnp.float32),
            pltpu.VMEM((BW,), jnp.float32),
            pltpu.VMEM((BW,), jnp.float32),
            pltpu.VMEM((L,), jnp.float32),
            pltpu.SemaphoreType.DMA,
            pltpu.SemaphoreType.DMA,
            pltpu.SemaphoreType.DMA,
            pltpu.SemaphoreType.DMA,
            pltpu.SemaphoreType.DMA,
            pltpu.SemaphoreType.DMA,
        ],
    )
    def k(uidx_hbm, fidx_hbm, uemb_hbm, ubias_hbm, femb_hbm, fbias_hbm,
          part_hbm, ub_hbm, fb_hbm,
          uidx_v, fidx_v, pu_v, pf_v, su_v, sf_v, urows_v, frows_v,
          ub_v, fb_v, part_v,
          sem_u0, sem_u1, sem_f0, sem_f1, sem_ub, sem_fb):
        wid = lax.axis_index("s") * NC + lax.axis_index("c")
        base = wid * BW
        pltpu.sync_copy(uidx_hbm.at[pl.ds(base, BW)], uidx_v)
        pltpu.sync_copy(fidx_hbm.at[pl.ds(base, BW)], fidx_v)
        cub = pltpu.async_copy(ubias_hbm.at[uidx_v], ub_v, sem_ub)
        cfb = pltpu.async_copy(fbias_hbm.at[fidx_v], fb_v, sem_fb)

        def idx_body(kk, _):
            o = kk * L
            u16 = uidx_v[pl.ds(o, L)]
            f16 = fidx_v[pl.ds(o, L)]
            pu_v[pl.ds(o, L)] = lax.shift_right_logical(u16, 1)
            pf_v[pl.ds(o, L)] = lax.shift_right_logical(f16, 1)
            su_v[pl.ds(o, L)] = lax.shift_left(jnp.bitwise_and(u16, 1), 6)
            sf_v[pl.ds(o, L)] = lax.shift_left(jnp.bitwise_and(f16, 1), 6)
            return 0

        lax.fori_loop(0, BW // L, idx_body, 0)

        sems_u = (sem_u0, sem_u1)
        sems_f = (sem_f0, sem_f1)

        def fire(c):
            b = c % 2
            cu = pltpu.async_copy(
                uemb_hbm.at[pu_v.at[pl.ds(c * CH, CH)]], urows_v.at[b],
                sems_u[b])
            cf = pltpu.async_copy(
                femb_hbm.at[pf_v.at[pl.ds(c * CH, CH)]], frows_v.at[b],
                sems_f[b])
            return cu, cf

        zero = jnp.zeros((L,), jnp.float32)
        accs = (zero, zero, zero, zero)
        pend = fire(0)
        for c in range(NCH):
            nxt = fire(c + 1) if c + 1 < NCH else None
            pend[0].wait()
            pend[1].wait()
            b = c % 2

            def grp_body(g, accs, _b=b, _c=c):
                a0, a1, a2, a3 = accs
                su16 = su_v[pl.ds(_c * CH + g * L, L)]
                sf16 = sf_v[pl.ds(_c * CH + g * L, L)]
                for j in range(L):
                    i = g * L + j
                    ou = su16[j]
                    of = sf16[j]
                    a0 = a0 + (urows_v[_b, i, pl.ds(ou + 0 * L, L)]
                               * frows_v[_b, i, pl.ds(of + 0 * L, L)])
                    a1 = a1 + (urows_v[_b, i, pl.ds(ou + 1 * L, L)]
                               * frows_v[_b, i, pl.ds(of + 1 * L, L)])
                    a2 = a2 + (urows_v[_b, i, pl.ds(ou + 2 * L, L)]
                               * frows_v[_b, i, pl.ds(of + 2 * L, L)])
                    a3 = a3 + (urows_v[_b, i, pl.ds(ou + 3 * L, L)]
                               * frows_v[_b, i, pl.ds(of + 3 * L, L)])
                return (a0, a1, a2, a3)

            accs = lax.fori_loop(0, CH // L, grp_body, accs)
            pend = nxt

        a0, a1, a2, a3 = accs
        part_v[...] = (a0 + a1) + (a2 + a3)
        pltpu.sync_copy(part_v, part_hbm.at[pl.ds(wid * L, L)])
        cub.wait()
        cfb.wait()
        pltpu.sync_copy(ub_v, ub_hbm.at[pl.ds(base, BW)])
        pltpu.sync_copy(fb_v, fb_hbm.at[pl.ds(base, BW)])

    return k(uidx, fidx, uemb2, user_bias, femb2, food_bias)


def _tc_finish(part, ub, fb):
    def body(p_ref, u_ref, f_ref, o_ref):
        s = jnp.sum(p_ref[...])
        o_ref[...] = jax.nn.sigmoid(u_ref[...] + f_ref[...] + s)

    return pl.pallas_call(
        body,
        out_shape=jax.ShapeDtypeStruct((128, 128), jnp.float32),
    )(part.reshape(4, 128), ub.reshape(128, 128), fb.reshape(128, 128))


def kernel(inputs, user_emb, user_bias, food_emb, food_bias):
    uidx = inputs[:, 0]
    fidx = inputs[:, 1]
    part, ub, fb = _sc_gather_partials(
        uidx, fidx,
        user_emb.reshape(NUM_PAIR_ROWS, 2 * D), user_bias.reshape(-1),
        food_emb.reshape(NUM_PAIR_ROWS, 2 * D), food_bias.reshape(-1))
    return _tc_finish(part, ub, fb).reshape(B, 1)


NUM_PAIR_ROWS = 500000
